# baseline (speedup 1.0000x reference)
import functools

import jax
import jax.numpy as jnp
from jax.experimental import pallas as pl
from jax.experimental.pallas import tpu as pltpu


def _gate_body(x_ref, w1_ref, b1_ref, w2_ref, b2_ref, o_ref, acc_ref,
               *, n_experts):
    h_idx = pl.program_id(1)
    hblk = jnp.tanh(
        jax.lax.dot_general(
            x_ref[...], w1_ref[...], (((1,), (0,)), ((), ())),
            preferred_element_type=jnp.float32,
        ) + b1_ref[...]
    )
    part = jax.lax.dot_general(
        hblk, w2_ref[...], (((1,), (0,)), ((), ())),
        preferred_element_type=jnp.float32,
    )

    @pl.when(h_idx == 0)
    def _init():
        acc_ref[...] = part + b2_ref[...]

    @pl.when(h_idx > 0)
    def _accum():
        acc_ref[...] += part

    @pl.when(h_idx == pl.num_programs(1) - 1)
    def _finalize():
        acc = acc_ref[...]
        mx = jnp.max(acc, axis=1, keepdims=True)
        iota = jax.lax.broadcasted_iota(jnp.int32, acc.shape, 1)
        idx = jnp.min(jnp.where(acc == mx, iota, n_experts), axis=1,
                      keepdims=True)
        o_ref[...] = (iota == idx).astype(jnp.float32)


def kernel(x, W1, b1, W2, b2):
    m, k = x.shape
    _, h = W1.shape
    e = W2.shape[1]
    bm = min(512, m)
    bh = min(512, h)
    b1r = b1.reshape(1, h)
    b2r = b2.reshape(1, e)
    grid = (m // bm, h // bh)
    return pl.pallas_call(
        functools.partial(_gate_body, n_experts=e),
        grid=grid,
        in_specs=[
            pl.BlockSpec((bm, k), lambda i, j: (i, 0)),
            pl.BlockSpec((k, bh), lambda i, j: (0, j)),
            pl.BlockSpec((1, bh), lambda i, j: (0, j)),
            pl.BlockSpec((bh, e), lambda i, j: (j, 0)),
            pl.BlockSpec((1, e), lambda i, j: (0, 0)),
        ],
        out_specs=pl.BlockSpec((bm, e), lambda i, j: (i, 0)),
        out_shape=jax.ShapeDtypeStruct((m, e), jnp.float32),
        scratch_shapes=[pltpu.VMEM((bm, e), jnp.float32)],
        compiler_params=pltpu.CompilerParams(
            dimension_semantics=("parallel", "arbitrary"),
        ),
    )(x, W1, b1r, W2, b2r)


# speedup vs baseline: 1.1203x; 1.1203x over previous
import functools

import jax
import jax.numpy as jnp
from jax.experimental import pallas as pl
from jax.experimental.pallas import tpu as pltpu


def _gate_body(x_ref, w1_ref, b1_ref, w2_ref, b2_ref, o_ref, acc_ref,
               *, n_experts):
    h_idx = pl.program_id(1)
    hblk = jnp.tanh(
        jax.lax.dot_general(
            x_ref[...], w1_ref[...], (((1,), (0,)), ((), ())),
            preferred_element_type=jnp.float32,
        ) + b1_ref[...]
    )
    part = jax.lax.dot_general(
        hblk, w2_ref[...], (((1,), (0,)), ((), ())),
        preferred_element_type=jnp.float32,
    )

    @pl.when(h_idx == 0)
    def _init():
        acc_ref[...] = part + b2_ref[...]

    @pl.when(h_idx > 0)
    def _accum():
        acc_ref[...] += part

    @pl.when(h_idx == pl.num_programs(1) - 1)
    def _finalize():
        acc = acc_ref[...]
        mx = jnp.max(acc, axis=1, keepdims=True)
        iota = jax.lax.broadcasted_iota(jnp.int32, acc.shape, 1)
        idx = jnp.min(jnp.where(acc == mx, iota, n_experts), axis=1,
                      keepdims=True)
        o_ref[...] = (iota == idx).astype(jnp.float32)


def kernel(x, W1, b1, W2, b2):
    m, k = x.shape
    _, h = W1.shape
    e = W2.shape[1]
    bm = min(2048, m)
    bh = min(512, h)
    x = x.astype(jnp.bfloat16)
    W1 = W1.astype(jnp.bfloat16)
    b1r = b1.reshape(1, h)
    b2r = b2.reshape(1, e)
    grid = (m // bm, h // bh)
    return pl.pallas_call(
        functools.partial(_gate_body, n_experts=e),
        grid=grid,
        in_specs=[
            pl.BlockSpec((bm, k), lambda i, j: (i, 0)),
            pl.BlockSpec((k, bh), lambda i, j: (0, j)),
            pl.BlockSpec((1, bh), lambda i, j: (0, j)),
            pl.BlockSpec((bh, e), lambda i, j: (j, 0)),
            pl.BlockSpec((1, e), lambda i, j: (0, 0)),
        ],
        out_specs=pl.BlockSpec((bm, e), lambda i, j: (i, 0)),
        out_shape=jax.ShapeDtypeStruct((m, e), jnp.float32),
        scratch_shapes=[pltpu.VMEM((bm, e), jnp.float32)],
        compiler_params=pltpu.CompilerParams(
            dimension_semantics=("parallel", "arbitrary"),
        ),
    )(x, W1, b1r, W2, b2r)
